# baseline (speedup 1.0000x reference)
import functools

import jax
import jax.numpy as jnp
from jax import lax
from jax.experimental import pallas as pl
from jax.experimental.pallas import tpu as pltpu
from jax.experimental.pallas import tpu_sc as plsc

NC = 2
NS = 16
NW = NC * NS
LANES = 16


def _make_emb_kernel(total, V, E, S, C):
    per_w = total // NW
    n_chunks = per_w // C
    mesh = plsc.VectorSubcoreMesh(
        core_axis_name="c", subcore_axis_name="s",
        num_cores=NC, num_subcores=NS)

    @functools.partial(
        pl.kernel,
        out_type=jax.ShapeDtypeStruct((total, E), jnp.float32),
        mesh=mesh,
        scratch_types=[
            pltpu.VMEM((per_w,), jnp.int32),
            pltpu.VMEM((S, E), jnp.float32),
            pltpu.VMEM((C, E), jnp.float32),
            pltpu.SemaphoreType.DMA,
        ],
    )
    def emb(ids_hbm, pos_hbm, table_hbm, out_hbm, idx_v, pos_v, buf_v, sem):
        wid = lax.axis_index("s") * NC + lax.axis_index("c")
        base = wid * per_w
        pltpu.sync_copy(ids_hbm.at[pl.ds(base, per_w)], idx_v)
        pltpu.sync_copy(pos_hbm, pos_v)

        def chunk_body(c, carry):
            row0 = c * C
            pltpu.async_copy(
                table_hbm.at[idx_v.at[pl.ds(row0, C)]], buf_v, sem).wait()

            def row_body(j, carry2):
                p = lax.rem(row0 + j, S)
                for k in range(E // LANES):
                    sl = pl.ds(k * LANES, LANES)
                    buf_v[j, sl] = buf_v[j, sl] + pos_v[p, sl]
                return carry2

            lax.fori_loop(0, C, row_body, 0, unroll=False)
            pltpu.sync_copy(buf_v, out_hbm.at[pl.ds(base + row0, C)])
            return carry

        lax.fori_loop(0, n_chunks, chunk_body, 0, unroll=False)

    return emb


@functools.partial(jax.jit, static_argnames=())
def kernel(input_ids, token_table, pos_table):
    B, S = input_ids.shape
    V, E = token_table.shape
    total = B * S
    ids = input_ids.reshape(total).astype(jnp.int32)
    C = 64
    assert total % NW == 0 and (total // NW) % C == 0 and E % LANES == 0
    emb = _make_emb_kernel(total, V, E, S, C)
    out = emb(ids, pos_table, token_table)
    return out.reshape(B, S, E)


# speedup vs baseline: 1.8157x; 1.8157x over previous
import functools

import jax
import jax.numpy as jnp
from jax import lax
from jax.experimental import pallas as pl
from jax.experimental.pallas import tpu as pltpu
from jax.experimental.pallas import tpu_sc as plsc

NC = 2
NS = 16
NW = NC * NS
LANES = 16
NBUF = 4


def _make_emb_kernel(total, V, E, S, C):
    per_w = total // NW
    n_chunks = per_w // C
    assert n_chunks % NBUF == 0
    mesh = plsc.VectorSubcoreMesh(
        core_axis_name="c", subcore_axis_name="s",
        num_cores=NC, num_subcores=NS)

    scratch = [pltpu.VMEM((per_w,), jnp.int32),
               pltpu.VMEM((S, E), jnp.float32)]
    scratch += [pltpu.VMEM((C, E), jnp.float32) for _ in range(NBUF)]
    scratch += [pltpu.SemaphoreType.DMA] * (2 * NBUF)

    @functools.partial(
        pl.kernel,
        out_type=jax.ShapeDtypeStruct((total, E), jnp.float32),
        mesh=mesh,
        scratch_types=scratch,
    )
    def emb(ids_hbm, pos_hbm, table_hbm, out_hbm, idx_v, pos_v, *rest):
        bufs = rest[:NBUF]
        gsems = rest[NBUF:2 * NBUF]
        ssems = rest[2 * NBUF:3 * NBUF]
        wid = lax.axis_index("s") * NC + lax.axis_index("c")
        base = wid * per_w
        pltpu.sync_copy(ids_hbm.at[pl.ds(base, per_w)], idx_v)
        pltpu.sync_copy(pos_hbm, pos_v)

        def gather(b, c):
            return pltpu.make_async_copy(
                table_hbm.at[idx_v.at[pl.ds(c * C, C)]], bufs[b], gsems[b])

        def store(b, c):
            return pltpu.make_async_copy(
                bufs[b], out_hbm.at[pl.ds(base + c * C, C)], ssems[b])

        gather(0, 0).start()
        gather(1, 1).start()

        def iter_body(i, carry):
            for b in range(NBUF):
                c = i * NBUF + b
                gather(b, c).wait()
                buf = bufs[b]

                @plsc.parallel_loop(0, C)
                def _(j):
                    p = lax.rem(c * C + j, S)
                    for k in range(E // LANES):
                        sl = pl.ds(k * LANES, LANES)
                        buf[j, sl] = buf[j, sl] + pos_v[p, sl]

                store(b, c).start()
                b2 = (b + 2) % NBUF
                cn = c + 2

                @pl.when(cn >= NBUF)
                def _():
                    store(b2, cn - NBUF).wait()

                @pl.when(cn < n_chunks)
                def _():
                    gather(b2, cn).start()
            return carry

        lax.fori_loop(0, n_chunks // NBUF, iter_body, 0, unroll=False)
        store((n_chunks - 2) % NBUF, n_chunks - 2).wait()
        store((n_chunks - 1) % NBUF, n_chunks - 1).wait()

    return emb


def kernel(input_ids, token_table, pos_table):
    B, S = input_ids.shape
    V, E = token_table.shape
    total = B * S
    ids = input_ids.reshape(total).astype(jnp.int32)
    C = 16
    assert total % NW == 0 and (total // NW) % C == 0 and E % LANES == 0
    emb = _make_emb_kernel(total, V, E, S, C)
    out = emb(ids, pos_table, token_table)
    return out.reshape(B, S, E)


# speedup vs baseline: 4.0991x; 2.2576x over previous
import functools

import jax
import jax.numpy as jnp
from jax import lax
from jax.experimental import pallas as pl
from jax.experimental.pallas import tpu as pltpu
from jax.experimental.pallas import tpu_sc as plsc

NC = 2
NS = 16
NW = NC * NS
LANES = 16
NBUF = 4


def _make_emb_kernel(total, V, E, S, B, C):
    per_w = total // NW
    n_chunks = per_w // C
    assert n_chunks % NBUF == 0
    mesh = plsc.VectorSubcoreMesh(
        core_axis_name="c", subcore_axis_name="s",
        num_cores=NC, num_subcores=NS)

    scratch = [pltpu.VMEM((per_w,), jnp.int32),
               pltpu.VMEM((S, E), jnp.float32)]
    scratch += [pltpu.VMEM((C, E), jnp.float32) for _ in range(NBUF)]
    scratch += [pltpu.SemaphoreType.DMA] * (2 * NBUF)

    @functools.partial(
        pl.kernel,
        out_type=jax.ShapeDtypeStruct((total, E), jnp.float32),
        mesh=mesh,
        scratch_types=scratch,
    )
    def emb(ids_hbm, pos_hbm, table_hbm, out_hbm, idx_v, pos_v, *rest):
        bufs = rest[:NBUF]
        gsems = rest[NBUF:2 * NBUF]
        ssems = rest[2 * NBUF:3 * NBUF]
        wid = lax.axis_index("s") * NC + lax.axis_index("c")
        base = wid * per_w
        pltpu.sync_copy(ids_hbm.at[pl.ds(base, per_w)], idx_v)
        pltpu.sync_copy(pos_hbm, pos_v)

        def gather(b, c):
            return pltpu.make_async_copy(
                table_hbm.at[idx_v.at[pl.ds(c * C, C)]], bufs[b], gsems[b])

        def store(b, c):
            return pltpu.make_async_copy(
                bufs[b], out_hbm.at[pl.ds(base + c * C, C)], ssems[b])

        gather(0, 0).start()
        gather(1, 1).start()

        def iter_body(i, carry):
            for b in range(NBUF):
                c = i * NBUF + b
                gather(b, c).wait()
                buf = bufs[b]

                @plsc.parallel_loop(0, C)
                def _(j):
                    p = lax.div(base + c * C + j, B)
                    for k in range(E // LANES):
                        sl = pl.ds(k * LANES, LANES)
                        buf[j, sl] = buf[j, sl] + pos_v[p, sl]

                store(b, c).start()
                b2 = (b + 2) % NBUF
                cn = c + 2

                @pl.when(cn >= NBUF)
                def _():
                    store(b2, cn - NBUF).wait()

                @pl.when(cn < n_chunks)
                def _():
                    gather(b2, cn).start()
            return carry

        lax.fori_loop(0, n_chunks // NBUF, iter_body, 0, unroll=False)
        store((n_chunks - 2) % NBUF, n_chunks - 2).wait()
        store((n_chunks - 1) % NBUF, n_chunks - 1).wait()

    return emb


def kernel(input_ids, token_table, pos_table):
    B, S = input_ids.shape
    V, E = token_table.shape
    total = B * S
    ids = input_ids.T.reshape(total).astype(jnp.int32)
    C = 16
    assert total % NW == 0 and (total // NW) % C == 0 and E % LANES == 0
    emb = _make_emb_kernel(total, V, E, S, B, C)
    out = emb(ids, pos_table, token_table)
    return out.reshape(S, B, E).transpose(1, 0, 2)


# speedup vs baseline: 6.4323x; 1.5692x over previous
import functools

import jax
import jax.numpy as jnp
from jax import lax
from jax.experimental import pallas as pl
from jax.experimental.pallas import tpu as pltpu
from jax.experimental.pallas import tpu_sc as plsc

NC = 2
NS = 16
NW = NC * NS
LANES = 16
NBUF = 4


def _make_emb_kernel(total, V, E, S, B, C):
    per_w = total // NW
    n_chunks = per_w // C
    assert n_chunks % NBUF == 0
    n_groups = E // LANES
    mesh = plsc.VectorSubcoreMesh(
        core_axis_name="c", subcore_axis_name="s",
        num_cores=NC, num_subcores=NS)

    scratch = [pltpu.VMEM((per_w,), jnp.int32),
               pltpu.VMEM((E,), jnp.float32)]
    scratch += [pltpu.VMEM((C, E), jnp.float32) for _ in range(NBUF)]
    scratch += [pltpu.SemaphoreType.DMA] * (2 * NBUF)

    @functools.partial(
        pl.kernel,
        out_type=jax.ShapeDtypeStruct((total, E), jnp.float32),
        mesh=mesh,
        scratch_types=scratch,
    )
    def emb(ids_hbm, pos_hbm, table_hbm, out_hbm, idx_v, posrow_v, *rest):
        bufs = rest[:NBUF]
        gsems = rest[NBUF:2 * NBUF]
        ssems = rest[2 * NBUF:3 * NBUF]
        wid = lax.axis_index("s") * NC + lax.axis_index("c")
        base = wid * per_w
        pltpu.sync_copy(ids_hbm.at[pl.ds(base, per_w)], idx_v)

        def gather(b, c):
            return pltpu.make_async_copy(
                table_hbm.at[idx_v.at[pl.ds(c * C, C)]], bufs[b], gsems[b])

        def store(b, c):
            return pltpu.make_async_copy(
                bufs[b], out_hbm.at[pl.ds(base + c * C, C)], ssems[b])

        gather(0, 0).start()
        gather(1, 1).start()

        def iter_body(i, p_prev):
            for b in range(NBUF):
                c = i * NBUF + b
                p = lax.div(base + c * C, B)

                @pl.when(p != p_prev)
                def _():
                    pltpu.sync_copy(pos_hbm.at[p], posrow_v)

                p_prev = p
                gather(b, c).wait()
                buf = bufs[b]
                for half in range(2):
                    k0 = half * (n_groups // 2)
                    pv = [posrow_v[pl.ds((k0 + k) * LANES, LANES)]
                          for k in range(n_groups // 2)]

                    @plsc.parallel_loop(0, C)
                    def _(j):
                        for k in range(n_groups // 2):
                            sl = pl.ds((k0 + k) * LANES, LANES)
                            buf[j, sl] = buf[j, sl] + pv[k]

                store(b, c).start()
                b2 = (b + 2) % NBUF
                cn = c + 2

                @pl.when(cn >= NBUF)
                def _():
                    store(b2, cn - NBUF).wait()

                @pl.when(cn < n_chunks)
                def _():
                    gather(b2, cn).start()
            return p_prev

        lax.fori_loop(0, n_chunks // NBUF, iter_body, jnp.int32(-1),
                      unroll=False)
        store((n_chunks - 2) % NBUF, n_chunks - 2).wait()
        store((n_chunks - 1) % NBUF, n_chunks - 1).wait()

    return emb


def kernel(input_ids, token_table, pos_table):
    B, S = input_ids.shape
    V, E = token_table.shape
    total = B * S
    ids = input_ids.T.reshape(total).astype(jnp.int32)
    C = 32
    assert total % NW == 0 and (total // NW) % C == 0 and E % LANES == 0
    assert B % C == 0
    emb = _make_emb_kernel(total, V, E, S, B, C)
    out = emb(ids, pos_table, token_table)
    return out.reshape(S, B, E).transpose(1, 0, 2)


# speedup vs baseline: 6.4344x; 1.0003x over previous
import functools

import jax
import jax.numpy as jnp
from jax import lax
from jax.experimental import pallas as pl
from jax.experimental.pallas import tpu as pltpu
from jax.experimental.pallas import tpu_sc as plsc

NC = 2
NS = 16
NW = NC * NS
LANES = 16
NBUF = 4


def _make_emb_kernel(total, V, E, S, B, C):
    per_w = total // NW
    n_chunks = per_w // C
    assert n_chunks % NBUF == 0
    n_groups = E // LANES
    mesh = plsc.VectorSubcoreMesh(
        core_axis_name="c", subcore_axis_name="s",
        num_cores=NC, num_subcores=NS)

    scratch = [pltpu.VMEM((per_w,), jnp.int32),
               pltpu.VMEM((E,), jnp.float32)]
    scratch += [pltpu.VMEM((C, E), jnp.float32) for _ in range(NBUF)]
    scratch += [pltpu.SemaphoreType.DMA] * (2 * NBUF)

    @functools.partial(
        pl.kernel,
        out_type=jax.ShapeDtypeStruct((total, E), jnp.float32),
        mesh=mesh,
        scratch_types=scratch,
    )
    def emb(ids_hbm, pos_hbm, table_hbm, out_hbm, idx_v, posrow_v, *rest):
        bufs = rest[:NBUF]
        gsems = rest[NBUF:2 * NBUF]
        ssems = rest[2 * NBUF:3 * NBUF]
        wid = lax.axis_index("s") * NC + lax.axis_index("c")
        base = wid * per_w
        pltpu.sync_copy(ids_hbm.at[pl.ds(base, per_w)], idx_v)

        def gather(b, c):
            return pltpu.make_async_copy(
                table_hbm.at[idx_v.at[pl.ds(c * C, C)]], bufs[b], gsems[b])

        def store(b, c):
            return pltpu.make_async_copy(
                bufs[b], out_hbm.at[pl.ds(base + c * C, C)], ssems[b])

        gather(0, 0).start()
        gather(1, 1).start()

        def iter_body(i, p_prev):
            for b in range(NBUF):
                c = i * NBUF + b
                p = lax.div(base + c * C, B)

                @pl.when(p != p_prev)
                def _():
                    pltpu.sync_copy(pos_hbm.at[p], posrow_v)

                p_prev = p
                gather(b, c).wait()
                buf = bufs[b]
                for half in range(2):
                    k0 = half * (n_groups // 2)
                    pv = [posrow_v[pl.ds((k0 + k) * LANES, LANES)]
                          for k in range(n_groups // 2)]

                    @plsc.parallel_loop(0, C, unroll=2)
                    def _(j):
                        for k in range(n_groups // 2):
                            sl = pl.ds((k0 + k) * LANES, LANES)
                            buf[j, sl] = buf[j, sl] + pv[k]

                store(b, c).start()
                b2 = (b + 2) % NBUF
                cn = c + 2

                @pl.when(cn >= NBUF)
                def _():
                    store(b2, cn - NBUF).wait()

                @pl.when(cn < n_chunks)
                def _():
                    gather(b2, cn).start()
            return p_prev

        lax.fori_loop(0, n_chunks // NBUF, iter_body, jnp.int32(-1),
                      unroll=False)
        store((n_chunks - 2) % NBUF, n_chunks - 2).wait()
        store((n_chunks - 1) % NBUF, n_chunks - 1).wait()

    return emb


def kernel(input_ids, token_table, pos_table):
    B, S = input_ids.shape
    V, E = token_table.shape
    total = B * S
    ids = input_ids.T.reshape(total).astype(jnp.int32)
    C = 32
    assert total % NW == 0 and (total // NW) % C == 0 and E % LANES == 0
    assert B % C == 0
    emb = _make_emb_kernel(total, V, E, S, B, C)
    out = emb(ids, pos_table, token_table)
    return out.reshape(S, B, E).transpose(1, 0, 2)
